# Initial kernel scaffold; baseline (speedup 1.0000x reference)
import jax
import jax.numpy as jnp
from jax.experimental import pallas as pl


def kernel(input):
    raise NotImplementedError("write your pallas kernel here")


# speedup vs baseline: 6.8316x; 6.8316x over previous
import jax
import jax.numpy as jnp
from jax import lax
from jax.experimental import pallas as pl
from jax.experimental.pallas import tpu as pltpu
from jax.experimental.pallas import tpu_sc as plsc

B, R, C = 32, 512, 512
W = R + C - 1
NROWS = B * R
NW = 32
ROWS_PER_W = NROWS // NW
CH = 16
NCH = ROWS_PER_W // CH
NGRP = W // 16


def _body(x_hbm, out_hbm, in_v, out_v):
    cid = lax.axis_index("c")
    sid = lax.axis_index("s")
    wid = sid * 2 + cid
    iota = lax.iota(jnp.int32, 16)

    def chunk(ci, _):
        row0 = wid * ROWS_PER_W + ci * CH
        pltpu.sync_copy(x_hbm.at[pl.ds(row0, CH), :], in_v)

        def row(i2, _):
            g = row0 + i2
            r = lax.bitwise_and(g, R - 1)
            s = (R - 1) - r
            rowv = jnp.full((16,), i2, dtype=jnp.int32)

            def grp(gi, _):
                j0 = gi * 16
                t = iota + (j0 - s)
                m = (t >= 0) & (t < C)
                idx = lax.bitwise_and(t, C - 1)
                v = plsc.load_gather(in_v, [rowv, idx])
                v = jnp.where(m, v, 0.0)
                out_v[i2, pl.ds(j0, 16)] = v
                return _

            lax.fori_loop(0, NGRP, grp, None)

            j0 = NGRP * 16
            t = iota + (j0 - s)
            m = (t >= 0) & (t < C)
            idx = lax.bitwise_and(t, C - 1)
            v = plsc.load_gather(in_v, [rowv, idx])
            v = jnp.where(m, v, 0.0)
            pos = jnp.minimum(iota + j0, W - 1)
            plsc.store_scatter(out_v, [rowv, pos], v, mask=iota < 15)
            return _

        lax.fori_loop(0, CH, row, None)
        pltpu.sync_copy(out_v, out_hbm.at[pl.ds(row0, CH), :])
        return _

    lax.fori_loop(0, NCH, chunk, None)


def kernel(input):
    xf = input.reshape(NROWS, C)
    f = pl.kernel(
        _body,
        out_type=jax.ShapeDtypeStruct((NROWS, W), jnp.float32),
        mesh=plsc.VectorSubcoreMesh(core_axis_name="c", subcore_axis_name="s"),
        compiler_params=pltpu.CompilerParams(needs_layout_passes=False),
        scratch_types=[
            pltpu.VMEM((CH, C), jnp.float32),
            pltpu.VMEM((CH, W), jnp.float32),
        ],
    )
    out = f(xf)
    return out.reshape(B, R, W)


# speedup vs baseline: 7.6803x; 1.1242x over previous
import jax
import jax.numpy as jnp
from jax import lax
from jax.experimental import pallas as pl
from jax.experimental.pallas import tpu as pltpu
from jax.experimental.pallas import tpu_sc as plsc

B, R, C = 32, 512, 512
W = R + C - 1
NROWS = B * R
NW = 32
ROWS_PER_W = NROWS // NW
CH = 16
NCH = ROWS_PER_W // CH
NG = C // 16
FLAT = CH * W
LIM = FLAT - 16


def _zero_buf(buf, zvec):
    def grp(i, _):
        buf[pl.ds(i * 16, 16)] = zvec
        return _
    lax.fori_loop(0, FLAT // 16, grp, None)


def _compute(ci, in_b, out_b, zvec, iota):
    def row(i2, _):
        r = ci * CH + i2
        s = (R - 1) - r
        a = i2 * W + s
        bf = lax.bitwise_and(a, ~15)
        d = a - bf
        rowv = jnp.full((16,), i2, dtype=jnp.int32)
        t0 = iota - d

        out_b[pl.ds(jnp.minimum(bf + (NG + 1) * 16, LIM), 16)] = zvec
        out_b[pl.ds(jnp.minimum(bf + (NG + 2) * 16, LIM), 16)] = zvec

        v = plsc.load_gather(in_b, [rowv, lax.bitwise_and(t0, C - 1)])
        out_b[pl.ds(bf, 16)] = jnp.where(t0 >= 0, v, 0.0)
        for g in range(1, NG):
            v = plsc.load_gather(in_b, [rowv, t0 + g * 16])
            out_b[pl.ds(bf + g * 16, 16)] = v
        t = t0 + NG * 16
        v = plsc.load_gather(in_b, [rowv, lax.bitwise_and(t, C - 1)])
        out_b[pl.ds(bf + NG * 16, 16)] = jnp.where(t < C, v, 0.0)
        return _

    lax.fori_loop(0, CH, row, None)


def _body(x_hbm, out_hbm, in_v, out_v):
    cid = lax.axis_index("c")
    sid = lax.axis_index("s")
    wid = sid * 2 + cid
    zvec = jnp.zeros((16,), jnp.float32)
    iota = lax.iota(jnp.int32, 16)
    _zero_buf(out_v, zvec)

    def chunk(ci, _):
        row0 = wid * ROWS_PER_W + ci * CH
        pltpu.sync_copy(x_hbm.at[pl.ds(row0, CH), :], in_v)
        _compute(ci, in_v, out_v, zvec, iota)
        pltpu.sync_copy(out_v, out_hbm.at[pl.ds(row0 * W, FLAT)])
        return _

    lax.fori_loop(0, NCH, chunk, None)


def kernel(input):
    xf = input.reshape(NROWS, C)
    f = pl.kernel(
        _body,
        out_type=jax.ShapeDtypeStruct((NROWS * W,), jnp.float32),
        mesh=plsc.VectorSubcoreMesh(core_axis_name="c", subcore_axis_name="s"),
        compiler_params=pltpu.CompilerParams(needs_layout_passes=False),
        scratch_types=[
            pltpu.VMEM((CH, C), jnp.float32),
            pltpu.VMEM((FLAT,), jnp.float32),
        ],
    )
    out = f(xf)
    return out.reshape(B, R, W)


# speedup vs baseline: 9.6220x; 1.2528x over previous
import jax
import jax.numpy as jnp
from jax import lax
from jax.experimental import pallas as pl
from jax.experimental.pallas import tpu as pltpu
from jax.experimental.pallas import tpu_sc as plsc

B, R, C = 32, 512, 512
W = R + C - 1
NROWS = B * R
NW = 32
ROWS_PER_W = NROWS // NW
CH = 16
NCH = ROWS_PER_W // CH
NG = C // 16
FLAT = CH * W
LIM = FLAT - 16


def _zero_buf(buf, zvec):
    def grp(i, _):
        buf[pl.ds(i * 16, 16)] = zvec
        return _
    lax.fori_loop(0, FLAT // 16, grp, None)


def _compute(ci, in_b, out_b, zvec, iota):
    def row(i2, _):
        r = ci * CH + i2
        s = (R - 1) - r
        a = i2 * W + s
        bf = lax.bitwise_and(a, ~15)
        d = a - bf
        rowv = jnp.full((16,), i2, dtype=jnp.int32)
        t0 = iota - d

        out_b[pl.ds(jnp.minimum(bf + (NG + 1) * 16, LIM), 16)] = zvec
        out_b[pl.ds(jnp.minimum(bf + (NG + 2) * 16, LIM), 16)] = zvec

        v = plsc.load_gather(in_b, [rowv, lax.bitwise_and(t0, C - 1)])
        out_b[pl.ds(bf, 16)] = jnp.where(t0 >= 0, v, 0.0)
        for g in range(1, NG):
            v = plsc.load_gather(in_b, [rowv, t0 + g * 16])
            out_b[pl.ds(bf + g * 16, 16)] = v
        t = t0 + NG * 16
        v = plsc.load_gather(in_b, [rowv, lax.bitwise_and(t, C - 1)])
        out_b[pl.ds(bf + NG * 16, 16)] = jnp.where(t < C, v, 0.0)
        return _

    lax.fori_loop(0, CH, row, None)


def _body(x_hbm, out_hbm, in0, in1, out0, out1, si0, si1, so0, so1):
    cid = lax.axis_index("c")
    sid = lax.axis_index("s")
    wid = sid * 2 + cid
    zvec = jnp.zeros((16,), jnp.float32)
    iota = lax.iota(jnp.int32, 16)
    _zero_buf(out0, zvec)
    _zero_buf(out1, zvec)

    def in_copy(ci, buf, sem):
        row0 = wid * ROWS_PER_W + ci * CH
        return pltpu.make_async_copy(x_hbm.at[pl.ds(row0, CH), :], buf, sem)

    def out_copy(ci, buf, sem):
        row0 = wid * ROWS_PER_W + ci * CH
        return pltpu.make_async_copy(buf, out_hbm.at[pl.ds(row0 * W, FLAT)],
                                     sem)

    in_copy(0, in0, si0).start()

    def loop(ci2, _):
        ciA = 2 * ci2
        ciB = ciA + 1
        in_copy(ciB, in1, si1).start()
        in_copy(ciA, in0, si0).wait()

        @pl.when(ci2 > 0)
        def _wa():
            out_copy(ciA - 2, out0, so0).wait()

        _compute(ciA, in0, out0, zvec, iota)
        out_copy(ciA, out0, so0).start()

        @pl.when(ci2 < NCH // 2 - 1)
        def _nb():
            in_copy(ciA + 2, in0, si0).start()

        in_copy(ciB, in1, si1).wait()

        @pl.when(ci2 > 0)
        def _wb():
            out_copy(ciB - 2, out1, so1).wait()

        _compute(ciB, in1, out1, zvec, iota)
        out_copy(ciB, out1, so1).start()
        return _

    lax.fori_loop(0, NCH // 2, loop, None)
    out_copy(NCH - 2, out0, so0).wait()
    out_copy(NCH - 1, out1, so1).wait()


def kernel(input):
    xf = input.reshape(NROWS, C)
    f = pl.kernel(
        _body,
        out_type=jax.ShapeDtypeStruct((NROWS * W,), jnp.float32),
        mesh=plsc.VectorSubcoreMesh(core_axis_name="c", subcore_axis_name="s"),
        compiler_params=pltpu.CompilerParams(needs_layout_passes=False),
        scratch_types=[
            pltpu.VMEM((CH, C), jnp.float32),
            pltpu.VMEM((CH, C), jnp.float32),
            pltpu.VMEM((FLAT,), jnp.float32),
            pltpu.VMEM((FLAT,), jnp.float32),
            pltpu.SemaphoreType.DMA,
            pltpu.SemaphoreType.DMA,
            pltpu.SemaphoreType.DMA,
            pltpu.SemaphoreType.DMA,
        ],
    )
    out = f(xf)
    return out.reshape(B, R, W)


# speedup vs baseline: 11.7873x; 1.2250x over previous
import jax
import jax.numpy as jnp
from jax import lax
from jax.experimental import pallas as pl
from jax.experimental.pallas import tpu as pltpu
from jax.experimental.pallas import tpu_sc as plsc

B, R, C = 32, 512, 512
W = R + C - 1
CH = 16
NCH = R // CH
NG = C // 16


def _zero_buf(buf, zvec):
    def row(i2, _):
        for g in range(W // 16):
            buf[i2, pl.ds(g * 16, 16)] = zvec
        buf[i2, pl.ds(W - 16, 16)] = zvec
        return _
    lax.fori_loop(0, CH, row, None)


def _compute(ci, in_b, out_b, zvec, iota):
    def row(i2, _):
        r = ci * CH + i2
        s = (R - 1) - r
        bf = lax.bitwise_and(s, ~15)
        d = s - bf
        rowv = jnp.full((16,), i2, dtype=jnp.int32)
        t0 = iota - d

        out_b[i2, pl.ds(jnp.minimum(bf + (NG + 1) * 16, W - 16), 16)] = zvec
        out_b[i2, pl.ds(jnp.minimum(bf + (NG + 2) * 16, W - 16), 16)] = zvec

        v = plsc.load_gather(in_b, [rowv, lax.bitwise_and(t0, C - 1)])
        out_b[i2, pl.ds(bf, 16)] = jnp.where(t0 >= 0, v, 0.0)
        for g in range(1, NG):
            v = plsc.load_gather(in_b, [rowv, t0 + g * 16])
            out_b[i2, pl.ds(bf + g * 16, 16)] = v
        t = t0 + NG * 16
        v = plsc.load_gather(in_b, [rowv, lax.bitwise_and(t, C - 1)])
        v = jnp.where(t < C, v, 0.0)
        pos = bf + NG * 16 + iota
        plsc.store_scatter(out_b, [rowv, jnp.minimum(pos, W - 1)], v,
                           mask=pos < W)
        return _

    lax.fori_loop(0, CH, row, None)


def _body(x_hbm, out_hbm, in0, in1, out0, out1, si0, si1, so0, so1):
    cid = lax.axis_index("c")
    sid = lax.axis_index("s")
    wid = sid * 2 + cid
    zvec = jnp.zeros((16,), jnp.float32)
    iota = lax.iota(jnp.int32, 16)
    _zero_buf(out0, zvec)
    _zero_buf(out1, zvec)

    def in_copy(ci, buf, sem):
        return pltpu.make_async_copy(
            x_hbm.at[wid, pl.ds(ci * CH, CH), :], buf, sem)

    def out_copy(ci, buf, sem):
        return pltpu.make_async_copy(
            buf, out_hbm.at[wid, pl.ds(ci * CH, CH), :], sem)

    in_copy(0, in0, si0).start()

    def loop(ci2, _):
        ciA = 2 * ci2
        ciB = ciA + 1
        in_copy(ciB, in1, si1).start()
        in_copy(ciA, in0, si0).wait()

        @pl.when(ci2 > 0)
        def _wa():
            out_copy(ciA - 2, out0, so0).wait()

        _compute(ciA, in0, out0, zvec, iota)
        out_copy(ciA, out0, so0).start()

        @pl.when(ci2 < NCH // 2 - 1)
        def _nb():
            in_copy(ciA + 2, in0, si0).start()

        in_copy(ciB, in1, si1).wait()

        @pl.when(ci2 > 0)
        def _wb():
            out_copy(ciB - 2, out1, so1).wait()

        _compute(ciB, in1, out1, zvec, iota)
        out_copy(ciB, out1, so1).start()
        return _

    lax.fori_loop(0, NCH // 2, loop, None)
    out_copy(NCH - 2, out0, so0).wait()
    out_copy(NCH - 1, out1, so1).wait()


def kernel(input):
    f = pl.kernel(
        _body,
        out_type=jax.ShapeDtypeStruct((B, R, W), jnp.float32),
        mesh=plsc.VectorSubcoreMesh(core_axis_name="c", subcore_axis_name="s"),
        compiler_params=pltpu.CompilerParams(needs_layout_passes=False),
        scratch_types=[
            pltpu.VMEM((CH, C), jnp.float32),
            pltpu.VMEM((CH, C), jnp.float32),
            pltpu.VMEM((CH, W), jnp.float32),
            pltpu.VMEM((CH, W), jnp.float32),
            pltpu.SemaphoreType.DMA,
            pltpu.SemaphoreType.DMA,
            pltpu.SemaphoreType.DMA,
            pltpu.SemaphoreType.DMA,
        ],
    )
    return f(input)


# speedup vs baseline: 11.9879x; 1.0170x over previous
import jax
import jax.numpy as jnp
from jax import lax
from jax.experimental import pallas as pl
from jax.experimental.pallas import tpu as pltpu
from jax.experimental.pallas import tpu_sc as plsc

B, R, C = 32, 512, 512
W = R + C - 1
CH = 32
NCH = R // CH
NG = C // 16


def _zero_buf(buf, zvec):
    def row(i2, _):
        for g in range(W // 16):
            buf[i2, pl.ds(g * 16, 16)] = zvec
        buf[i2, pl.ds(W - 16, 16)] = zvec
        return _
    lax.fori_loop(0, CH, row, None)


def _compute(ci, in_b, out_b, zvec, iota):
    def row(i2, _):
        r = ci * CH + i2
        s = (R - 1) - r
        bf = lax.bitwise_and(s, ~15)
        d = s - bf
        rowv = jnp.full((16,), i2, dtype=jnp.int32)
        t0 = iota - d

        for k in range(1, 5):
            out_b[i2, pl.ds(jnp.minimum(bf + (NG + k) * 16, W - 16), 16)] = zvec

        v = plsc.load_gather(in_b, [rowv, lax.bitwise_and(t0, C - 1)])
        out_b[i2, pl.ds(bf, 16)] = jnp.where(t0 >= 0, v, 0.0)
        for g in range(1, NG):
            v = plsc.load_gather(in_b, [rowv, t0 + g * 16])
            out_b[i2, pl.ds(bf + g * 16, 16)] = v
        t = t0 + NG * 16
        v = plsc.load_gather(in_b, [rowv, lax.bitwise_and(t, C - 1)])
        v = jnp.where(t < C, v, 0.0)
        pos = bf + NG * 16 + iota
        plsc.store_scatter(out_b, [rowv, jnp.minimum(pos, W - 1)], v,
                           mask=pos < W)
        return _

    lax.fori_loop(0, CH, row, None)


def _body(x_hbm, out_hbm, in0, in1, out0, out1, si0, si1, so0, so1):
    cid = lax.axis_index("c")
    sid = lax.axis_index("s")
    wid = sid * 2 + cid
    zvec = jnp.zeros((16,), jnp.float32)
    iota = lax.iota(jnp.int32, 16)
    _zero_buf(out0, zvec)
    _zero_buf(out1, zvec)

    def in_copy(ci, buf, sem):
        return pltpu.make_async_copy(
            x_hbm.at[wid, pl.ds(ci * CH, CH), :], buf, sem)

    def out_copy(ci, buf, sem):
        return pltpu.make_async_copy(
            buf, out_hbm.at[wid, pl.ds(ci * CH, CH), :], sem)

    in_copy(0, in0, si0).start()

    def loop(ci2, _):
        ciA = 2 * ci2
        ciB = ciA + 1
        in_copy(ciB, in1, si1).start()
        in_copy(ciA, in0, si0).wait()

        @pl.when(ci2 > 0)
        def _wa():
            out_copy(ciA - 2, out0, so0).wait()

        _compute(ciA, in0, out0, zvec, iota)
        out_copy(ciA, out0, so0).start()

        @pl.when(ci2 < NCH // 2 - 1)
        def _nb():
            in_copy(ciA + 2, in0, si0).start()

        in_copy(ciB, in1, si1).wait()

        @pl.when(ci2 > 0)
        def _wb():
            out_copy(ciB - 2, out1, so1).wait()

        _compute(ciB, in1, out1, zvec, iota)
        out_copy(ciB, out1, so1).start()
        return _

    lax.fori_loop(0, NCH // 2, loop, None)
    out_copy(NCH - 2, out0, so0).wait()
    out_copy(NCH - 1, out1, so1).wait()


def kernel(input):
    f = pl.kernel(
        _body,
        out_type=jax.ShapeDtypeStruct((B, R, W), jnp.float32),
        mesh=plsc.VectorSubcoreMesh(core_axis_name="c", subcore_axis_name="s"),
        compiler_params=pltpu.CompilerParams(needs_layout_passes=False),
        scratch_types=[
            pltpu.VMEM((CH, C), jnp.float32),
            pltpu.VMEM((CH, C), jnp.float32),
            pltpu.VMEM((CH, W), jnp.float32),
            pltpu.VMEM((CH, W), jnp.float32),
            pltpu.SemaphoreType.DMA,
            pltpu.SemaphoreType.DMA,
            pltpu.SemaphoreType.DMA,
            pltpu.SemaphoreType.DMA,
        ],
    )
    return f(input)
